# Initial kernel scaffold; baseline (speedup 1.0000x reference)
import jax
import jax.numpy as jnp
from jax.experimental import pallas as pl


def kernel(projections_1, projections_2, feature_queue):
    raise NotImplementedError("write your pallas kernel here")


# speedup vs baseline: 1.4453x; 1.4453x over previous
import functools

import jax
import jax.numpy as jnp
from jax import lax
from jax.experimental import pallas as pl
from jax.experimental.pallas import tpu as pltpu
from jax.experimental.pallas import tpu_sc as plsc

B = 1024
D = 128
Q = 100000
QBLK = 1024
NSTEPS = (Q + QBLK - 1) // QBLK
TEMPERATURE = 0.1
NEG = -3.0e38
INT_BIG = 2**31 - 1

SC_NC = 2
SC_NS = 16
SC_NW = SC_NC * SC_NS
ROWS_PER_W = (2 * B) // SC_NW


def _sim_argmax_body(pnorm_ref, fq_ref, idx_ref, newq_ref, runmax_ref):
    i = pl.program_id(0)

    @pl.when(i == 0)
    def _init():
        runmax_ref[...] = jnp.full((2 * B, 1), NEG, jnp.float32)
        idx_ref[...] = jnp.zeros((2 * B, 1), jnp.int32)

    fqb = fq_ref[...]
    scores = lax.dot_general(pnorm_ref[...], fqb, (((1,), (1,)), ((), ())),
                             preferred_element_type=jnp.float32)
    col = i * QBLK + lax.broadcasted_iota(jnp.int32, (2 * B, QBLK), 1)
    scores = jnp.where(col < Q, scores, NEG)
    bmax = jnp.max(scores, axis=1, keepdims=True)
    bidx = jnp.min(jnp.where(scores == bmax, col, INT_BIG), axis=1,
                   keepdims=True)
    better = bmax > runmax_ref[...]
    idx_ref[...] = jnp.where(better, bidx, idx_ref[...])
    bmax_bf = bmax.astype(jnp.bfloat16).astype(jnp.float32)
    runmax_ref[...] = jnp.where(better, bmax_bf, runmax_ref[...])

    @pl.when(i < NSTEPS - 1)
    def _shift_copy():
        newq_ref[...] = fqb

    @pl.when(i == NSTEPS - 1)
    def _queue_head():
        newq_ref[...] = pnorm_ref[pl.ds(0, B), :]


_sim_argmax = pl.pallas_call(
    _sim_argmax_body,
    grid=(NSTEPS,),
    in_specs=[
        pl.BlockSpec((2 * B, D), lambda i: (0, 0)),
        pl.BlockSpec((QBLK, D), lambda i: (i, 0)),
    ],
    out_specs=[
        pl.BlockSpec((2 * B, 1), lambda i: (0, 0)),
        pl.BlockSpec((QBLK, D), lambda i: ((i + 1) % NSTEPS, 0)),
    ],
    out_shape=[
        jax.ShapeDtypeStruct((2 * B, 1), jnp.int32),
        jax.ShapeDtypeStruct((Q, D), jnp.float32),
    ],
    scratch_shapes=[pltpu.VMEM((2 * B, 1), jnp.float32)],
    compiler_params=pltpu.CompilerParams(
        dimension_semantics=("arbitrary",)),
)


_SC_GATHER_CACHE = []


def _sc_gather(table, idx):
    if not _SC_GATHER_CACHE:
        @functools.partial(
            pl.kernel,
            mesh=plsc.VectorSubcoreMesh(core_axis_name="c",
                                        subcore_axis_name="s"),
            out_type=jax.ShapeDtypeStruct((2 * B, D), jnp.float32),
            scratch_types=[
                pltpu.VMEM((ROWS_PER_W,), jnp.int32),
                pltpu.VMEM((ROWS_PER_W, D), jnp.float32),
                pltpu.SemaphoreType.DMA,
            ],
        )
        def gather_k(table_hbm, idx_hbm, out_hbm, idx_v, rows_v, sem):
            wid = lax.axis_index("s") * SC_NC + lax.axis_index("c")
            base = wid * ROWS_PER_W
            pltpu.sync_copy(idx_hbm.at[pl.ds(base, ROWS_PER_W)], idx_v)
            pltpu.async_copy(table_hbm.at[idx_v], rows_v, sem).wait()
            pltpu.sync_copy(rows_v, out_hbm.at[pl.ds(base, ROWS_PER_W)])

        _SC_GATHER_CACHE.append(gather_k)
    return _SC_GATHER_CACHE[0](table, idx)


def _loss_body(nn_ref, pother_ref, loss_ref):
    g = pl.program_id(0)
    odd = (g % 2) == 1
    nn = nn_ref[...]
    po = pother_ref[...]
    a = jnp.where(odd, po, nn)
    b = jnp.where(odd, nn, po)
    s = lax.dot_general(a, b, (((1,), (1,)), ((), ())),
                        preferred_element_type=jnp.float32)
    s = s * (1.0 / TEMPERATURE)
    r = lax.broadcasted_iota(jnp.int32, (B, B), 0)
    c = lax.broadcasted_iota(jnp.int32, (B, B), 1)
    diag = jnp.sum(jnp.where(r == c, s, 0.0), axis=1, keepdims=True)
    m = jnp.max(s, axis=1, keepdims=True)
    lse = jnp.log(jnp.sum(jnp.exp(s - m), axis=1, keepdims=True)) + m
    loss_ref[...] = lse - diag


_loss_call = pl.pallas_call(
    _loss_body,
    grid=(4,),
    in_specs=[
        pl.BlockSpec((B, D), lambda g: (g // 2, 0)),
        pl.BlockSpec((B, D), lambda g: (1 - g // 2, 0)),
    ],
    out_specs=pl.BlockSpec((B, 1), lambda g: (g, 0)),
    out_shape=jax.ShapeDtypeStruct((4 * B, 1), jnp.float32),
    compiler_params=pltpu.CompilerParams(
        dimension_semantics=("arbitrary",)),
)


def _l2_normalize(x):
    sq = jnp.sum(x * x, axis=1, keepdims=True)
    return x * lax.rsqrt(jnp.maximum(sq, 1e-12))


def kernel(projections_1, projections_2, feature_queue):
    pnorm = jnp.concatenate([_l2_normalize(projections_1),
                             _l2_normalize(projections_2)], axis=0)
    idx, new_queue = _sim_argmax(pnorm, feature_queue)
    nn = _sc_gather(feature_queue, idx.reshape(2 * B))
    loss = _loss_call(nn, pnorm)
    return loss.reshape(4 * B), new_queue
